# baseline (speedup 1.0000x reference)
import functools

import jax
import jax.numpy as jnp
from jax import lax
from jax.experimental import pallas as pl
from jax.experimental.pallas import tpu as pltpu
from jax.experimental.pallas import tpu_sc as plsc

_B = 16384
_D = 64
_V_ITEM = 1000000
_V_BRAND = 100000
_V_CAT = 1000
_TEXT_DIM = 768

_CHUNK_TC = 1
_CW = _CHUNK_TC * 128
_SKEW = _CW + 5


@functools.cache
def _make_scan_gather(v):
    info = plsc.get_sparse_core_info()
    nc, ns = info.num_cores, info.num_subcores
    nw = nc * ns
    tcols = -(-v // 128)
    tpw = -(-tcols // nw)
    cpw = -(-tpw // _CHUNK_TC)
    cpw += cpw % 2
    hcap = _B + 16

    mesh = plsc.VectorSubcoreMesh(core_axis_name="c", subcore_axis_name="s")

    @functools.partial(
        pl.kernel,
        mesh=mesh,
        compiler_params=pltpu.CompilerParams(needs_layout_passes=False),
        out_type=jax.ShapeDtypeStruct((_B + 16, _D), jnp.float32),
        scratch_types=[
            pltpu.VMEM((_B,), jnp.int32),
            pltpu.VMEM((hcap,), jnp.int32),
            pltpu.VMEM((hcap,), jnp.int32),
            pltpu.VMEM((hcap,), jnp.int32),
            pltpu.VMEM((hcap,), jnp.int32),
            pltpu.VMEM((_D, _SKEW), jnp.float32),
            pltpu.VMEM((_D, _SKEW), jnp.float32),
            pltpu.VMEM((16, _D), jnp.float32),
            pltpu.SemaphoreType.DMA,
            pltpu.SemaphoreType.DMA,
            pltpu.SemaphoreType.DMA,
        ],
    )
    def scan_gather(idx_hbm, tab_t, out,
                    iall, hidx, hpos, cidx, cpos, buf0, buf1, rowb,
                    semf0, semf1, semw):
        wid = lax.axis_index("s") * nc + lax.axis_index("c")
        tc_start = wid * tpw
        lo = tc_start * 128
        hi = jnp.minimum((tc_start + tpw) * 128, v)

        pltpu.sync_copy(idx_hbm, iall)

        lanes = lax.iota(jnp.int32, 16)

        def scan_body(g, off):
            vec = iall[pl.ds(g * 16, 16)]
            m = (vec >= lo) & (vec < hi)
            cs = plsc.cumsum(m.astype(jnp.int32))
            tgt = jnp.where(m, off + cs - 1, hcap - 16)
            plsc.store_scatter(hidx, [tgt], vec)
            plsc.store_scatter(hpos, [tgt], g * 16 + lanes)
            return off + cs[15]

        nhits = lax.fori_loop(0, _B // 16, scan_body, jnp.int32(0))

        def chunk_base(q):
            return jnp.minimum(tc_start + q * _CHUNK_TC,
                               tcols - _CHUNK_TC) * 128

        def fetch(q, buf, sem):
            return pltpu.async_copy(
                tab_t.at[:, pl.ds(chunk_base(q), _CW)],
                buf.at[:, pl.ds(0, _CW)], sem)

        def process(q, buf):
            base = chunk_base(q)
            def cscan(t, coff):
                hv = hidx[pl.ds(t * 16, 16)]
                hp = hpos[pl.ds(t * 16, 16)]
                inb = (t * 16 + lanes) < nhits
                m = (hv >= base) & (hv < base + _CW) & inb
                cs = plsc.cumsum(m.astype(jnp.int32))
                tgt = jnp.where(m, coff + cs - 1, hcap - 16)
                plsc.store_scatter(cidx, [tgt], hv)
                plsc.store_scatter(cpos, [tgt], hp)
                return coff + cs[15]

            cnt = lax.fori_loop(0, (nhits + 15) // 16, cscan, jnp.int32(0))
            cidx[pl.ds(cnt, 16)] = jnp.full((16,), base, jnp.int32)
            cpos[pl.ds(cnt, 16)] = jnp.full((16,), _B, jnp.int32)

            def group(g, _):
                cv = cidx[pl.ds(g * 16, 16)] - base
                cp = cpos[pl.ds(g * 16, 16)]
                for j in range(16):
                    col = cv[j]
                    for g4 in range(4):
                        vals = plsc.load_gather(
                            buf, [g4 * 16 + lanes,
                                  jnp.full((16,), col, jnp.int32)])
                        rowb.at[j][pl.ds(g4 * 16, 16)] = vals
                for j in range(16):
                    pltpu.async_copy(rowb.at[pl.ds(j, 1)],
                                     out.at[pl.ds(cp[j], 1)], semw)
                pltpu.make_async_copy(
                    out.at[pl.ds(0, 16)], rowb, semw).wait()
                return _

            lax.fori_loop(0, (cnt + 15) // 16, group, jnp.int32(0))

        f0 = fetch(0, buf0, semf0)
        f1 = fetch(1, buf1, semf1)
        del f0, f1

        def ring(q2, _):
            q = q2 * 2
            pltpu.make_async_copy(
                tab_t.at[:, pl.ds(0, _CW)],
                buf0.at[:, pl.ds(0, _CW)], semf0).wait()
            process(q, buf0)

            @pl.when(q2 + 1 < cpw // 2)
            def _f0():
                fetch(q + 2, buf0, semf0)

            pltpu.make_async_copy(
                tab_t.at[:, pl.ds(0, _CW)],
                buf1.at[:, pl.ds(0, _CW)], semf1).wait()
            process(q + 1, buf1)

            @pl.when(q2 + 1 < cpw // 2)
            def _f1():
                fetch(q + 3, buf1, semf1)

            return _

        lax.fori_loop(0, cpw // 2, ring, jnp.int32(0))

    return scan_gather


def _row_gather_body(n_tables, nc, bpw, args):
    idx_hbms = args[:n_tables]
    tabs = args[n_tables:2 * n_tables]
    outs = args[2 * n_tables:3 * n_tables]
    iv, rows, sem = args[3 * n_tables:]
    wid = lax.axis_index("s") * nc + lax.axis_index("c")
    base = wid * bpw
    for idx_hbm, tab, out in zip(idx_hbms, tabs, outs):
        pltpu.sync_copy(idx_hbm.at[pl.ds(base, bpw)], iv)

        def body(k, _):
            vec = iv[pl.ds(k * 16, 16)]
            for j in range(16):
                pltpu.async_copy(tab.at[pl.ds(vec[j], 1)],
                                 rows.at[pl.ds(k * 16 + j, 1)], sem)
            return _

        lax.fori_loop(0, bpw // 16, body, 0)
        pltpu.make_async_copy(tab.at[pl.ds(0, bpw)], rows, sem).wait()
        pltpu.sync_copy(rows, out.at[pl.ds(base, bpw)])


@functools.cache
def _make_row_gather(n_tables):
    info = plsc.get_sparse_core_info()
    nc, ns = info.num_cores, info.num_subcores
    bpw = _B // (nc * ns)

    mesh = plsc.VectorSubcoreMesh(core_axis_name="c", subcore_axis_name="s")

    @functools.partial(
        pl.kernel,
        mesh=mesh,
        out_type=[jax.ShapeDtypeStruct((_B, _D), jnp.float32)] * n_tables,
        scratch_types=[
            pltpu.VMEM((bpw,), jnp.int32),
            pltpu.VMEM((bpw, _D), jnp.float32),
            pltpu.SemaphoreType.DMA,
        ],
    )
    def gather(*args):
        _row_gather_body(n_tables, nc, bpw, args)

    return gather


def _mlp_body(text_ref, ei_ref, eb_ref, ec_ref,
              wt_ref, bt_ref, w1_ref, b1_ref, w2_ref, b2_ref, w3_ref, b3_ref,
              out_ref):
    e_text = (jnp.dot(text_ref[...], wt_ref[...],
                      preferred_element_type=jnp.float32) + bt_ref[...])
    x = jnp.concatenate([ei_ref[...], eb_ref[...], ec_ref[...], e_text],
                        axis=-1)
    h = jnp.maximum(jnp.dot(x, w1_ref[...],
                            preferred_element_type=jnp.float32) + b1_ref[...],
                    0.0)
    h = jnp.maximum(jnp.dot(h, w2_ref[...],
                            preferred_element_type=jnp.float32) + b2_ref[...],
                    0.0)
    o = (jnp.dot(h, w3_ref[...], preferred_element_type=jnp.float32)
         + b3_ref[...])
    n = jnp.maximum(jnp.sqrt(jnp.sum(o * o, axis=1, keepdims=True)), 1e-12)
    out_ref[...] = o / n


def kernel(item_id, brand, category, text_features, emb_item_id, emb_brand,
           emb_category, W_text, b_text, W1, b1, W2, b2, W3, b3):
    ii = jnp.clip(item_id, 0, _V_ITEM - 1)
    bb = jnp.clip(brand, 0, _V_BRAND - 1)
    cc = jnp.clip(category, 0, _V_CAT - 1)

    e_item_pad = _make_scan_gather(_V_ITEM)(ii, emb_item_id.T)
    e_item = e_item_pad[:_B]
    e_brand, e_cat = _make_row_gather(2)(bb, cc, emb_brand, emb_category)

    blk = 1024
    grid = (_B // blk,)

    def b_spec(w):
        return pl.BlockSpec((blk, w), lambda i: (i, 0))

    def w_spec(shape):
        return pl.BlockSpec(shape, lambda i: (0, 0))

    out = pl.pallas_call(
        _mlp_body,
        grid=grid,
        in_specs=[
            b_spec(_TEXT_DIM),
            b_spec(_D), b_spec(_D), b_spec(_D),
            w_spec((_TEXT_DIM, _D)), w_spec((1, _D)),
            w_spec((4 * _D, 256)), w_spec((1, 256)),
            w_spec((256, 128)), w_spec((1, 128)),
            w_spec((128, _D)), w_spec((1, _D)),
        ],
        out_specs=b_spec(_D),
        out_shape=jax.ShapeDtypeStruct((_B, _D), jnp.float32),
    )(text_features, e_item, e_brand, e_cat,
      W_text, b_text.reshape(1, _D), W1, b1.reshape(1, 256),
      W2, b2.reshape(1, 128), W3, b3.reshape(1, _D))
    return out


# speedup vs baseline: 1.0037x; 1.0037x over previous
import functools

import jax
import jax.numpy as jnp
from jax import lax
from jax.experimental import pallas as pl
from jax.experimental.pallas import tpu as pltpu
from jax.experimental.pallas import tpu_sc as plsc

_B = 16384
_D = 64
_V_ITEM = 1000000
_V_BRAND = 100000
_V_CAT = 1000
_TEXT_DIM = 768

_CHUNK_TC = 1
_CW = _CHUNK_TC * 128
_SKEW = _CW + 5


@functools.cache
def _make_scan_gather(v):
    info = plsc.get_sparse_core_info()
    nc, ns = info.num_cores, info.num_subcores
    nw = nc * ns
    tcols = -(-v // 128)
    tpw = -(-tcols // nw)
    cpw = -(-tpw // _CHUNK_TC)
    cpw += cpw % 2
    hcap = _B + 16

    mesh = plsc.VectorSubcoreMesh(core_axis_name="c", subcore_axis_name="s")

    @functools.partial(
        pl.kernel,
        mesh=mesh,
        compiler_params=pltpu.CompilerParams(needs_layout_passes=False),
        out_type=jax.ShapeDtypeStruct((_B + 16, _D), jnp.float32),
        scratch_types=[
            pltpu.VMEM((_B,), jnp.int32),
            pltpu.VMEM((hcap,), jnp.int32),
            pltpu.VMEM((hcap,), jnp.int32),
            pltpu.VMEM((hcap,), jnp.int32),
            pltpu.VMEM((hcap,), jnp.int32),
            pltpu.VMEM((_D, _SKEW), jnp.float32),
            pltpu.VMEM((_D, _SKEW), jnp.float32),
            pltpu.VMEM((16, _D), jnp.float32),
            pltpu.SemaphoreType.DMA,
            pltpu.SemaphoreType.DMA,
            pltpu.SemaphoreType.DMA,
        ],
    )
    def scan_gather(idx_hbm, tab_t, out,
                    iall, hidx, hpos, cidx, cpos, buf0, buf1, rowb,
                    semf0, semf1, semw):
        wid = lax.axis_index("s") * nc + lax.axis_index("c")
        tc_start = wid * tpw
        lo = tc_start * 128
        hi = jnp.minimum((tc_start + tpw) * 128, v)

        pltpu.sync_copy(idx_hbm, iall)

        lanes = lax.iota(jnp.int32, 16)

        def scan_body(g, off):
            vec = iall[pl.ds(g * 16, 16)]
            m = (vec >= lo) & (vec < hi)
            cs = plsc.cumsum(m.astype(jnp.int32))
            tgt = jnp.where(m, off + cs - 1, hcap - 16)
            plsc.store_scatter(hidx, [tgt], vec)
            plsc.store_scatter(hpos, [tgt], g * 16 + lanes)
            return off + cs[15]

        nhits = lax.fori_loop(0, _B // 16, scan_body, jnp.int32(0))

        def chunk_base(q):
            return jnp.minimum(tc_start + q * _CHUNK_TC,
                               tcols - _CHUNK_TC) * 128

        def fetch(q, buf, sem):
            base = chunk_base(q)
            for a in range(8):
                pltpu.async_copy(
                    tab_t.at[pl.ds(a * 8, 8), pl.ds(base, _CW)],
                    buf.at[pl.ds(a * 8, 8), pl.ds(0, _CW)], sem)

        def process(q, buf):
            base = chunk_base(q)
            def cscan(t, coff):
                hv = hidx[pl.ds(t * 16, 16)]
                hp = hpos[pl.ds(t * 16, 16)]
                inb = (t * 16 + lanes) < nhits
                m = (hv >= base) & (hv < base + _CW) & inb
                cs = plsc.cumsum(m.astype(jnp.int32))
                tgt = jnp.where(m, coff + cs - 1, hcap - 16)
                plsc.store_scatter(cidx, [tgt], hv)
                plsc.store_scatter(cpos, [tgt], hp)
                return coff + cs[15]

            cnt = lax.fori_loop(0, (nhits + 15) // 16, cscan, jnp.int32(0))
            cidx[pl.ds(cnt, 16)] = jnp.full((16,), base, jnp.int32)
            cpos[pl.ds(cnt, 16)] = jnp.full((16,), _B, jnp.int32)

            def group(g, _):
                cv = cidx[pl.ds(g * 16, 16)] - base
                cp = cpos[pl.ds(g * 16, 16)]
                for j in range(16):
                    col = cv[j]
                    for g4 in range(4):
                        vals = plsc.load_gather(
                            buf, [g4 * 16 + lanes,
                                  jnp.full((16,), col, jnp.int32)])
                        rowb.at[j][pl.ds(g4 * 16, 16)] = vals
                for j in range(16):
                    pltpu.async_copy(rowb.at[pl.ds(j, 1)],
                                     out.at[pl.ds(cp[j], 1)], semw)
                pltpu.make_async_copy(
                    out.at[pl.ds(0, 16)], rowb, semw).wait()
                return _

            lax.fori_loop(0, (cnt + 15) // 16, group, jnp.int32(0))

        fetch(0, buf0, semf0)
        fetch(1, buf1, semf1)

        def ring(q2, _):
            q = q2 * 2
            pltpu.make_async_copy(
                tab_t.at[:, pl.ds(0, _CW)],
                buf0.at[:, pl.ds(0, _CW)], semf0).wait()
            process(q, buf0)

            @pl.when(q2 + 1 < cpw // 2)
            def _f0():
                fetch(q + 2, buf0, semf0)

            pltpu.make_async_copy(
                tab_t.at[:, pl.ds(0, _CW)],
                buf1.at[:, pl.ds(0, _CW)], semf1).wait()
            process(q + 1, buf1)

            @pl.when(q2 + 1 < cpw // 2)
            def _f1():
                fetch(q + 3, buf1, semf1)

            return _

        lax.fori_loop(0, cpw // 2, ring, jnp.int32(0))

    return scan_gather


def _row_gather_body(n_tables, nc, bpw, args):
    idx_hbms = args[:n_tables]
    tabs = args[n_tables:2 * n_tables]
    outs = args[2 * n_tables:3 * n_tables]
    iv, rows, sem = args[3 * n_tables:]
    wid = lax.axis_index("s") * nc + lax.axis_index("c")
    base = wid * bpw
    for idx_hbm, tab, out in zip(idx_hbms, tabs, outs):
        pltpu.sync_copy(idx_hbm.at[pl.ds(base, bpw)], iv)

        def body(k, _):
            vec = iv[pl.ds(k * 16, 16)]
            for j in range(16):
                pltpu.async_copy(tab.at[pl.ds(vec[j], 1)],
                                 rows.at[pl.ds(k * 16 + j, 1)], sem)
            return _

        lax.fori_loop(0, bpw // 16, body, 0)
        pltpu.make_async_copy(tab.at[pl.ds(0, bpw)], rows, sem).wait()
        pltpu.sync_copy(rows, out.at[pl.ds(base, bpw)])


@functools.cache
def _make_row_gather(n_tables):
    info = plsc.get_sparse_core_info()
    nc, ns = info.num_cores, info.num_subcores
    bpw = _B // (nc * ns)

    mesh = plsc.VectorSubcoreMesh(core_axis_name="c", subcore_axis_name="s")

    @functools.partial(
        pl.kernel,
        mesh=mesh,
        out_type=[jax.ShapeDtypeStruct((_B, _D), jnp.float32)] * n_tables,
        scratch_types=[
            pltpu.VMEM((bpw,), jnp.int32),
            pltpu.VMEM((bpw, _D), jnp.float32),
            pltpu.SemaphoreType.DMA,
        ],
    )
    def gather(*args):
        _row_gather_body(n_tables, nc, bpw, args)

    return gather


def _mlp_body(text_ref, ei_ref, eb_ref, ec_ref,
              wt_ref, bt_ref, w1_ref, b1_ref, w2_ref, b2_ref, w3_ref, b3_ref,
              out_ref):
    e_text = (jnp.dot(text_ref[...], wt_ref[...],
                      preferred_element_type=jnp.float32) + bt_ref[...])
    x = jnp.concatenate([ei_ref[...], eb_ref[...], ec_ref[...], e_text],
                        axis=-1)
    h = jnp.maximum(jnp.dot(x, w1_ref[...],
                            preferred_element_type=jnp.float32) + b1_ref[...],
                    0.0)
    h = jnp.maximum(jnp.dot(h, w2_ref[...],
                            preferred_element_type=jnp.float32) + b2_ref[...],
                    0.0)
    o = (jnp.dot(h, w3_ref[...], preferred_element_type=jnp.float32)
         + b3_ref[...])
    n = jnp.maximum(jnp.sqrt(jnp.sum(o * o, axis=1, keepdims=True)), 1e-12)
    out_ref[...] = o / n


def kernel(item_id, brand, category, text_features, emb_item_id, emb_brand,
           emb_category, W_text, b_text, W1, b1, W2, b2, W3, b3):
    ii = jnp.clip(item_id, 0, _V_ITEM - 1)
    bb = jnp.clip(brand, 0, _V_BRAND - 1)
    cc = jnp.clip(category, 0, _V_CAT - 1)

    e_item_pad = _make_scan_gather(_V_ITEM)(ii, emb_item_id.T)
    e_item = e_item_pad[:_B]
    e_brand, e_cat = _make_row_gather(2)(bb, cc, emb_brand, emb_category)

    blk = 1024
    grid = (_B // blk,)

    def b_spec(w):
        return pl.BlockSpec((blk, w), lambda i: (i, 0))

    def w_spec(shape):
        return pl.BlockSpec(shape, lambda i: (0, 0))

    out = pl.pallas_call(
        _mlp_body,
        grid=grid,
        in_specs=[
            b_spec(_TEXT_DIM),
            b_spec(_D), b_spec(_D), b_spec(_D),
            w_spec((_TEXT_DIM, _D)), w_spec((1, _D)),
            w_spec((4 * _D, 256)), w_spec((1, 256)),
            w_spec((256, 128)), w_spec((1, 128)),
            w_spec((128, _D)), w_spec((1, _D)),
        ],
        out_specs=b_spec(_D),
        out_shape=jax.ShapeDtypeStruct((_B, _D), jnp.float32),
    )(text_features, e_item, e_brand, e_cat,
      W_text, b_text.reshape(1, _D), W1, b1.reshape(1, 256),
      W2, b2.reshape(1, 128), W3, b3.reshape(1, _D))
    return out


# speedup vs baseline: 8.8847x; 8.8518x over previous
import functools

import jax
import jax.numpy as jnp
from jax import lax
from jax.experimental import pallas as pl
from jax.experimental.pallas import tpu as pltpu
from jax.experimental.pallas import tpu_sc as plsc

_B = 16384
_D = 64
_V_ITEM = 1000000
_V_BRAND = 100000
_V_CAT = 1000
_TEXT_DIM = 768


def _gather_body(n_tables, nc, bpw, args):
    idx_hbms = args[:n_tables]
    tabs = args[n_tables:2 * n_tables]
    outs = args[2 * n_tables:3 * n_tables]
    iv, rows, sem = args[3 * n_tables:]
    wid = lax.axis_index("s") * nc + lax.axis_index("c")
    base = wid * bpw
    for idx_hbm, tab, out in zip(idx_hbms, tabs, outs):
        pltpu.sync_copy(idx_hbm.at[pl.ds(base, bpw)], iv)

        def body(k, _):
            vec = iv[pl.ds(k * 16, 16)]
            for j in range(16):
                pltpu.async_copy(tab.at[pl.ds(vec[j], 1)],
                                 rows.at[pl.ds(k * 16 + j, 1)], sem)
            return _

        lax.fori_loop(0, bpw // 16, body, 0)
        pltpu.make_async_copy(tab.at[pl.ds(0, bpw)], rows, sem).wait()
        pltpu.sync_copy(rows, out.at[pl.ds(base, bpw)])


@functools.cache
def _make_gather(n_tables):
    info = plsc.get_sparse_core_info()
    nc, ns = info.num_cores, info.num_subcores
    bpw = _B // (nc * ns)

    mesh = plsc.VectorSubcoreMesh(core_axis_name="c", subcore_axis_name="s")

    @functools.partial(
        pl.kernel,
        mesh=mesh,
        out_type=[jax.ShapeDtypeStruct((_B, _D), jnp.float32)] * n_tables,
        scratch_types=[
            pltpu.VMEM((bpw,), jnp.int32),
            pltpu.VMEM((bpw, _D), jnp.float32),
            pltpu.SemaphoreType.DMA,
        ],
    )
    def gather(*args):
        _gather_body(n_tables, nc, bpw, args)

    return gather


def _mlp_body(text_ref, ei_ref, eb_ref, ec_ref,
              wt_ref, bt_ref, w1_ref, b1_ref, w2_ref, b2_ref, w3_ref, b3_ref,
              out_ref):
    e_text = (jnp.dot(text_ref[...], wt_ref[...],
                      preferred_element_type=jnp.float32) + bt_ref[...])
    x = jnp.concatenate([ei_ref[...], eb_ref[...], ec_ref[...], e_text],
                        axis=-1)
    h = jnp.maximum(jnp.dot(x, w1_ref[...],
                            preferred_element_type=jnp.float32) + b1_ref[...],
                    0.0)
    h = jnp.maximum(jnp.dot(h, w2_ref[...],
                            preferred_element_type=jnp.float32) + b2_ref[...],
                    0.0)
    o = (jnp.dot(h, w3_ref[...], preferred_element_type=jnp.float32)
         + b3_ref[...])
    n = jnp.maximum(jnp.sqrt(jnp.sum(o * o, axis=1, keepdims=True)), 1e-12)
    out_ref[...] = o / n


def kernel(item_id, brand, category, text_features, emb_item_id, emb_brand,
           emb_category, W_text, b_text, W1, b1, W2, b2, W3, b3):
    ii = jnp.clip(item_id, 0, _V_ITEM - 1)
    bb = jnp.clip(brand, 0, _V_BRAND - 1)
    cc = jnp.clip(category, 0, _V_CAT - 1)

    e_brand, e_cat = _make_gather(2)(bb, cc, emb_brand, emb_category)
    (e_item,) = _make_gather(1)(ii, emb_item_id)

    blk = 1024
    grid = (_B // blk,)

    def b_spec(w):
        return pl.BlockSpec((blk, w), lambda i: (i, 0))

    def w_spec(shape):
        return pl.BlockSpec(shape, lambda i: (0, 0))

    out = pl.pallas_call(
        _mlp_body,
        grid=grid,
        in_specs=[
            b_spec(_TEXT_DIM),
            b_spec(_D), b_spec(_D), b_spec(_D),
            w_spec((_TEXT_DIM, _D)), w_spec((1, _D)),
            w_spec((4 * _D, 256)), w_spec((1, 256)),
            w_spec((256, 128)), w_spec((1, 128)),
            w_spec((128, _D)), w_spec((1, _D)),
        ],
        out_specs=b_spec(_D),
        out_shape=jax.ShapeDtypeStruct((_B, _D), jnp.float32),
    )(text_features, e_item, e_brand, e_cat,
      W_text, b_text.reshape(1, _D), W1, b1.reshape(1, 256),
      W2, b2.reshape(1, 128), W3, b3.reshape(1, _D))
    return out


# speedup vs baseline: 8.9796x; 1.0107x over previous
import functools

import jax
import jax.numpy as jnp
from jax import lax
from jax.experimental import pallas as pl
from jax.experimental.pallas import tpu as pltpu
from jax.experimental.pallas import tpu_sc as plsc

_B = 16384
_D = 64
_V_ITEM = 1000000
_V_BRAND = 100000
_V_CAT = 1000
_TEXT_DIM = 768


def _gather_body(n_tables, nc, bpw, args):
    idx_hbms = args[:n_tables]
    tabs = args[n_tables:2 * n_tables]
    outs = args[2 * n_tables:3 * n_tables]
    iv, rows, sem = args[3 * n_tables:]
    wid = lax.axis_index("s") * nc + lax.axis_index("c")
    base = wid * bpw
    for idx_hbm, tab, out in zip(idx_hbms, tabs, outs):
        pltpu.sync_copy(idx_hbm.at[pl.ds(base, bpw)], iv)

        def body(k, _):
            vec = iv[pl.ds(k * 16, 16)]
            for j in range(16):
                pltpu.async_copy(tab.at[pl.ds(vec[j], 1)],
                                 rows.at[pl.ds(k * 16 + j, 1)], sem)
            return _

        lax.fori_loop(0, bpw // 16, body, 0)
        pltpu.make_async_copy(tab.at[pl.ds(0, bpw)], rows, sem).wait()
        pltpu.sync_copy(rows, out.at[pl.ds(base, bpw)])


@functools.cache
def _make_gather(n_tables):
    info = plsc.get_sparse_core_info()
    nc, ns = info.num_cores, info.num_subcores
    bpw = _B // (nc * ns)

    mesh = plsc.VectorSubcoreMesh(core_axis_name="c", subcore_axis_name="s")

    @functools.partial(
        pl.kernel,
        mesh=mesh,
        out_type=[jax.ShapeDtypeStruct((_B, _D), jnp.float32)] * n_tables,
        scratch_types=[
            pltpu.VMEM((bpw,), jnp.int32),
            pltpu.VMEM((bpw, _D), jnp.float32),
            pltpu.SemaphoreType.DMA,
        ],
    )
    def gather(*args):
        _gather_body(n_tables, nc, bpw, args)

    return gather


def _mlp_body(text_ref, ei_ref, eb_ref, ec_ref,
              wt_ref, bt_ref, w1_ref, b1_ref, w2_ref, b2_ref, w3_ref, b3_ref,
              out_ref):
    e_text = (jnp.dot(text_ref[...], wt_ref[...],
                      preferred_element_type=jnp.float32) + bt_ref[...])
    x = jnp.concatenate([ei_ref[...], eb_ref[...], ec_ref[...], e_text],
                        axis=-1)
    h = jnp.maximum(jnp.dot(x, w1_ref[...],
                            preferred_element_type=jnp.float32) + b1_ref[...],
                    0.0)
    h = jnp.maximum(jnp.dot(h, w2_ref[...],
                            preferred_element_type=jnp.float32) + b2_ref[...],
                    0.0)
    o = (jnp.dot(h, w3_ref[...], preferred_element_type=jnp.float32)
         + b3_ref[...])
    n = jnp.maximum(jnp.sqrt(jnp.sum(o * o, axis=1, keepdims=True)), 1e-12)
    out_ref[...] = o / n


def kernel(item_id, brand, category, text_features, emb_item_id, emb_brand,
           emb_category, W_text, b_text, W1, b1, W2, b2, W3, b3):
    ii = jnp.clip(item_id, 0, _V_ITEM - 1)
    bb = jnp.clip(brand, 0, _V_BRAND - 1)
    cc = jnp.clip(category, 0, _V_CAT - 1)

    e_brand, e_cat = _make_gather(2)(bb, cc, emb_brand, emb_category)
    (e_item,) = _make_gather(1)(ii, emb_item_id)

    blk = 2048
    grid = (_B // blk,)

    def b_spec(w):
        return pl.BlockSpec((blk, w), lambda i: (i, 0))

    def w_spec(shape):
        return pl.BlockSpec(shape, lambda i: (0, 0))

    out = pl.pallas_call(
        _mlp_body,
        grid=grid,
        in_specs=[
            b_spec(_TEXT_DIM),
            b_spec(_D), b_spec(_D), b_spec(_D),
            w_spec((_TEXT_DIM, _D)), w_spec((1, _D)),
            w_spec((4 * _D, 256)), w_spec((1, 256)),
            w_spec((256, 128)), w_spec((1, 128)),
            w_spec((128, _D)), w_spec((1, _D)),
        ],
        out_specs=b_spec(_D),
        out_shape=jax.ShapeDtypeStruct((_B, _D), jnp.float32),
    )(text_features, e_item, e_brand, e_cat,
      W_text, b_text.reshape(1, _D), W1, b1.reshape(1, 256),
      W2, b2.reshape(1, 128), W3, b3.reshape(1, _D))
    return out
